# Initial kernel scaffold; baseline (speedup 1.0000x reference)
import jax
import jax.numpy as jnp
from jax.experimental import pallas as pl


def kernel(nums, emb, W, b, k):
    raise NotImplementedError("write your pallas kernel here")


# speedup vs baseline: 1.5582x; 1.5582x over previous
import functools

import jax
import jax.numpy as jnp
from jax import lax
from jax.experimental import pallas as pl
from jax.experimental.pallas import tpu as pltpu
from jax.experimental.pallas import tpu_sc as plsc

DOC_LEN = 16384
DIMS = 128
CLASSES = 1000

NC = 2
NS = 16
NW = NC * NS
PER_W = DOC_LEN // NW
CHUNK = 128
NCHUNK = PER_W // CHUNK
LANES = 16
NVREG = DIMS // LANES
PAD_C = 1024


def _sc_body(nums_hbm, emb_hbm, out_hbm, idx_v, rows_v, acc_v, sem_a, sem_b):
    wid = lax.axis_index("s") * NC + lax.axis_index("c")
    pltpu.sync_copy(nums_hbm.at[wid], idx_v)
    sems = (sem_a, sem_b)
    cp = pltpu.async_copy(emb_hbm.at[idx_v.at[0]], rows_v.at[0], sems[0])
    accs = tuple(jnp.zeros((LANES,), jnp.float32) for _ in range(NVREG))
    for ch in range(NCHUNK):
        nxt = None
        if ch + 1 < NCHUNK:
            nxt = pltpu.async_copy(
                emb_hbm.at[idx_v.at[ch + 1]], rows_v.at[ch + 1],
                sems[(ch + 1) % 2])
        cp.wait()

        def row_body(i, a, _ch=ch):
            return tuple(
                a[j] + rows_v[_ch, i, pl.ds(j * LANES, LANES)]
                for j in range(NVREG))

        accs = lax.fori_loop(0, CHUNK, row_body, accs)
        cp = nxt
    for j in range(NVREG):
        acc_v[pl.ds(j * LANES, LANES)] = accs[j]
    pltpu.sync_copy(acc_v, out_hbm.at[wid])


_sc_gather_sum = functools.partial(
    pl.kernel,
    mesh=plsc.VectorSubcoreMesh(core_axis_name="c", subcore_axis_name="s"),
    out_type=jax.ShapeDtypeStruct((NW, DIMS), jnp.float32),
    scratch_types=[
        pltpu.VMEM((NCHUNK, CHUNK), jnp.int32),
        pltpu.VMEM((NCHUNK, CHUNK, DIMS), jnp.float32),
        pltpu.VMEM((DIMS,), jnp.float32),
        pltpu.SemaphoreType.DMA,
        pltpu.SemaphoreType.DMA,
    ],
)(_sc_body)


def _tc_body(k_ref, part_ref, w_ref, b_ref, out_ref):
    doc = jnp.sum(part_ref[...], axis=0, keepdims=True)
    logits = jnp.dot(doc, w_ref[...],
                     preferred_element_type=jnp.float32) + b_ref[...]
    m = jnp.max(logits)
    lse = jnp.log(jnp.sum(jnp.exp(logits - m))) + m
    col = lax.broadcasted_iota(jnp.int32, (1, PAD_C), 1)
    sel = jnp.sum(jnp.where(col == k_ref[0], logits, 0.0))
    out_ref[0, 0] = lse - sel


def _tc_tail(karr, partials, w_pad, b_pad):
    return pl.pallas_call(
        _tc_body,
        out_shape=jax.ShapeDtypeStruct((1, 1), jnp.float32),
        in_specs=[
            pl.BlockSpec(memory_space=pltpu.SMEM),
            pl.BlockSpec(memory_space=pltpu.VMEM),
            pl.BlockSpec(memory_space=pltpu.VMEM),
            pl.BlockSpec(memory_space=pltpu.VMEM),
        ],
        out_specs=pl.BlockSpec(memory_space=pltpu.SMEM),
    )(karr, partials, w_pad, b_pad)


def kernel(nums, emb, W, b, k):
    nums3 = nums.reshape(NW, NCHUNK, CHUNK).astype(jnp.int32)
    partials = _sc_gather_sum(nums3, emb)
    c = W.shape[1]
    w_pad = jnp.pad(W, ((0, 0), (0, PAD_C - c)))
    b_pad = jnp.pad(b, (0, PAD_C - c),
                    constant_values=-1e30).reshape(1, PAD_C)
    karr = jnp.asarray(k, jnp.int32).reshape(1)
    loss = _tc_tail(karr, partials, w_pad, b_pad)
    return loss[0, 0]
